# baseline (speedup 1.0000x reference)
import functools

import jax
import jax.numpy as jnp
from jax import lax
from jax.experimental import pallas as pl
from jax.experimental.pallas import tpu as pltpu
from jax.experimental.pallas import tpu_sc as plsc

_N = 10000
_E = 320000
_H = 128
_NC = 18
_G = 64

_NCORE = 2
_NSUB = 16
_NW = _NCORE * _NSUB
_EPT = _E // _NW
_K = 80
_NCHUNK = _EPT // _K
_KA = 80
_NPAD = 10240
_RPT = _NPAD // _NSUB
_ZR = 128

_BM = 2000
_CH = 2000
_NBLK = _N // _CH

_mesh = plsc.VectorSubcoreMesh(core_axis_name="c", subcore_axis_name="s")


@functools.partial(
    pl.kernel,
    out_type=(jax.ShapeDtypeStruct((_NPAD, 16), jnp.float32),
              jax.ShapeDtypeStruct((_NPAD, 16), jnp.float32)),
    mesh=_mesh,
    scratch_types=[
        pltpu.VMEM_SHARED((_NPAD, 16), jnp.float32),
        pltpu.VMEM((_K,), jnp.int32),
        pltpu.VMEM((_K, 16), jnp.float32),
        pltpu.VMEM((_RPT, 16), jnp.float32),
    ],
)
def _deg_kernel(dst_hbm, c0_hbm, c1_hbm, acc, dstb, onesb, zb):
    cid = lax.axis_index("c")
    sid = lax.axis_index("s")
    row0 = sid * _RPT

    def orow(i, c):
        onesb[i, :] = jnp.ones((16,), jnp.float32)
        return c
    lax.fori_loop(0, _K, orow, 0)

    def zrow(i, c):
        zb[i, :] = jnp.zeros((16,), jnp.float32)
        return c
    lax.fori_loop(0, _RPT, zrow, 0)
    pltpu.sync_copy(zb, acc.at[pl.ds(row0, _RPT), :])
    plsc.subcore_barrier()

    base = (cid * _NSUB + sid) * _EPT

    def step(k, c):
        off = pl.multiple_of(base + k * _K, 8)
        pltpu.sync_copy(dst_hbm.at[pl.ds(off, _K)], dstb)
        pltpu.sync_copy(onesb, acc.at[dstb], add=True)
        return c
    lax.fori_loop(0, _NCHUNK, step, 0)
    plsc.subcore_barrier()

    @pl.when(cid == 0)
    def _():
        pltpu.sync_copy(acc.at[pl.ds(row0, _RPT), :], c0_hbm.at[pl.ds(row0, _RPT), :])

    @pl.when(cid == 1)
    def _():
        pltpu.sync_copy(acc.at[pl.ds(row0, _RPT), :], c1_hbm.at[pl.ds(row0, _RPT), :])


_NBUF = 3
_NCAP = 126
_NGRP = _NCAP // _NBUF
_EPTP = _NCAP * _KA

_AGG_SCRATCH = [
    pltpu.VMEM_SHARED((_NPAD, _H), jnp.float32),
    pltpu.VMEM((2, _NBUF, _KA), jnp.int32),
    pltpu.VMEM((2, _NBUF, _KA), jnp.int32),
    pltpu.VMEM((_KA, _H), jnp.float32),
    pltpu.VMEM((_KA, _H), jnp.float32),
    pltpu.VMEM((_KA, _H), jnp.float32),
    pltpu.VMEM((_ZR, _H), jnp.float32),
    pltpu.SemaphoreType.DMA,
    pltpu.SemaphoreType.DMA,
]


def _agg_body(ap_hbm, src_hbm, dst_hbm, p0_hbm, p1_hbm, acc,
              ebs, ebd, rows0, rows1, rows2, zb, semg, semi):
    cid = lax.axis_index("c")
    sid = lax.axis_index("s")
    row0 = sid * _RPT
    wid = cid * _NSUB + sid
    rows = (rows0, rows1, rows2)

    pltpu.async_copy(src_hbm.at[wid, 0], ebs.at[0], semi)
    pltpu.async_copy(dst_hbm.at[wid, 0], ebd.at[0], semi)

    def zrow(i, c):
        for j in range(8):
            zb[i, pl.ds(j * 16, 16)] = jnp.zeros((16,), jnp.float32)
        return c
    lax.fori_loop(0, _ZR, zrow, 0)
    for t in range(_RPT // _ZR):
        pltpu.sync_copy(zb, acc.at[pl.ds(row0 + t * _ZR, _ZR), :])
    pltpu.make_async_copy(src_hbm.at[wid, 0], ebs.at[0], semi).wait()
    pltpu.make_async_copy(dst_hbm.at[wid, 0], ebd.at[0], semi).wait()
    plsc.subcore_barrier()

    def group(m, c):
        p = lax.rem(m, 2)

        @pl.when(m + 1 < _NGRP)
        def _():
            pltpu.async_copy(src_hbm.at[wid, m + 1], ebs.at[1 - p], semi)
            pltpu.async_copy(dst_hbm.at[wid, m + 1], ebd.at[1 - p], semi)

        descs = [
            pltpu.async_copy(ap_hbm.at[ebs.at[p, b]], rows[b], semg)
            for b in range(_NBUF)
        ]
        for b in range(_NBUF):
            descs[b].wait()
            pltpu.sync_copy(rows[b], acc.at[ebd.at[p, b]], add=True)

        @pl.when(m + 1 < _NGRP)
        def _():
            pltpu.make_async_copy(src_hbm.at[wid, m + 1], ebs.at[1 - p], semi).wait()
            pltpu.make_async_copy(dst_hbm.at[wid, m + 1], ebd.at[1 - p], semi).wait()
        return c
    lax.fori_loop(0, _NGRP, group, 0)
    plsc.subcore_barrier()

    @pl.when(cid == 0)
    def _():
        pltpu.sync_copy(acc.at[pl.ds(row0, _RPT), :], p0_hbm.at[pl.ds(row0, _RPT), :])

    @pl.when(cid == 1)
    def _():
        pltpu.sync_copy(acc.at[pl.ds(row0, _RPT), :], p1_hbm.at[pl.ds(row0, _RPT), :])


_agg_kernel = functools.partial(
    pl.kernel,
    out_type=(jax.ShapeDtypeStruct((_NPAD, _H), jnp.float32),
              jax.ShapeDtypeStruct((_NPAD, _H), jnp.float32)),
    mesh=_mesh,
    scratch_types=_AGG_SCRATCH,
)(_agg_body)


def _dinv_of(c0, c1):
    return lax.rsqrt(c0[:, 0:1] + c1[:, 0:1] + 1.0)


def _mm1_body(x_ref, w_ref, c0_ref, c1_ref, o_ref):
    dinv = _dinv_of(c0_ref[...], c1_ref[...])
    o_ref[...] = dinv * jnp.dot(x_ref[...], w_ref[...],
                                preferred_element_type=jnp.float32)


def _mm_first(x, W, c0, c1):
    return pl.pallas_call(
        _mm1_body,
        grid=(_N // _BM,),
        in_specs=[
            pl.BlockSpec((_BM, _H), lambda i: (i, 0)),
            pl.BlockSpec((_H, _H), lambda i: (0, 0)),
            pl.BlockSpec((_BM, 16), lambda i: (i, 0)),
            pl.BlockSpec((_BM, 16), lambda i: (i, 0)),
        ],
        out_specs=pl.BlockSpec((_BM, _H), lambda i: (i, 0)),
        out_shape=jax.ShapeDtypeStruct((_N, _H), jnp.float32),
    )(x, W, c0, c1)


def _mm_mid_body(p0_ref, p1_ref, ap_ref, c0_ref, c1_ref, b_ref, w_ref, o_ref):
    dinv = _dinv_of(c0_ref[...], c1_ref[...])
    h = dinv * (p0_ref[...] + p1_ref[...] + ap_ref[...]) + b_ref[...]
    h = jnp.maximum(h, 0.0)
    o_ref[...] = dinv * jnp.dot(h, w_ref[...], preferred_element_type=jnp.float32)


def _mm_mid(p0, p1, ap, c0, c1, b_row, W):
    return pl.pallas_call(
        _mm_mid_body,
        grid=(_N // _BM,),
        in_specs=[
            pl.BlockSpec((_BM, _H), lambda i: (i, 0)),
            pl.BlockSpec((_BM, _H), lambda i: (i, 0)),
            pl.BlockSpec((_BM, _H), lambda i: (i, 0)),
            pl.BlockSpec((_BM, 16), lambda i: (i, 0)),
            pl.BlockSpec((_BM, 16), lambda i: (i, 0)),
            pl.BlockSpec((1, _H), lambda i: (0, 0)),
            pl.BlockSpec((_H, _H), lambda i: (0, 0)),
        ],
        out_specs=pl.BlockSpec((_BM, _H), lambda i: (i, 0)),
        out_shape=jax.ShapeDtypeStruct((_N, _H), jnp.float32),
    )(p0, p1, ap, c0, c1, b_row, W)


def _pool_body(p0_ref, p1_ref, ap_ref, c0_ref, c1_ref, b_ref, bt_ref, cpad_ref,
               mx_ref, it_ref, o_ref, sums, cnt):
    i = pl.program_id(0)

    @pl.when(i == 0)
    def _():
        sums[...] = jnp.zeros_like(sums)
        cnt[...] = jnp.zeros_like(cnt)

    dinv = _dinv_of(c0_ref[...], c1_ref[...])
    h = dinv * (p0_ref[...] + p1_ref[...] + ap_ref[...]) + b_ref[...]
    bt = bt_ref[0]
    gi = lax.broadcasted_iota(jnp.int32, (_G, _CH), 0)
    s = (bt == gi).astype(jnp.float32)
    sums[...] += jnp.dot(s, h, preferred_element_type=jnp.float32)
    cnt[...] += jnp.sum(s, axis=1, keepdims=True)

    @pl.when(i == _NBLK - 1)
    def _():
        g = sums[...] / jnp.maximum(cnt[...], 1.0)
        cp = cpad_ref[...]
        cross = jnp.dot(g, cp, preferred_element_type=jnp.float32)
        cn2 = jnp.sum(cp * cp, axis=0, keepdims=True)
        gn2 = jnp.sum(g * g, axis=1, keepdims=True)
        d2 = jnp.maximum(gn2 + cn2 - 2.0 * cross, 0.0)
        dmin2 = jnp.minimum(d2[:, :64], d2[:, 64:])
        dist = jnp.sqrt(dmin2)
        lane64 = lax.broadcasted_iota(jnp.int32, (_G, 64), 1)
        md = jnp.min(jnp.where(lane64 < _NC, dist, 1e30), axis=1, keepdims=True)
        soft = 1.0 / (1.0 + jnp.exp(-(mx_ref[...] - md) * it_ref[...]))
        dist128 = jnp.concatenate([dist, dist], axis=1)
        lane = lax.broadcasted_iota(jnp.int32, (_G, _H), 1)
        o_ref[...] = jnp.where(lane < _NC, -dist128,
                               jnp.where(lane == _NC, soft, 0.0))


def _pool_head(p0, p1, ap, c0, c1, b_row, batch3, cpadT, mx_row, it_row):
    return pl.pallas_call(
        _pool_body,
        grid=(_NBLK,),
        in_specs=[
            pl.BlockSpec((_CH, _H), lambda i: (i, 0)),
            pl.BlockSpec((_CH, _H), lambda i: (i, 0)),
            pl.BlockSpec((_CH, _H), lambda i: (i, 0)),
            pl.BlockSpec((_CH, 16), lambda i: (i, 0)),
            pl.BlockSpec((_CH, 16), lambda i: (i, 0)),
            pl.BlockSpec((1, _H), lambda i: (0, 0)),
            pl.BlockSpec((1, 1, _CH), lambda i: (i, 0, 0)),
            pl.BlockSpec((_H, _H), lambda i: (0, 0)),
            pl.BlockSpec((1, _H), lambda i: (0, 0)),
            pl.BlockSpec((1, _H), lambda i: (0, 0)),
        ],
        out_specs=pl.BlockSpec((_G, _H), lambda i: (0, 0)),
        out_shape=jax.ShapeDtypeStruct((_G, _H), jnp.float32),
        scratch_shapes=[
            pltpu.VMEM((_G, _H), jnp.float32),
            pltpu.VMEM((_G, _H), jnp.float32),
        ],
    )(p0, p1, ap, c0, c1, b_row, batch3, cpadT, mx_row, it_row)


def kernel(x, edge_index, batch, W1, b1, W2, b2, W3, b3, centroids,
           std_scale, ac_temp, running_mean, running_var):
    src = edge_index[0]
    dst = edge_index[1]
    src3 = jnp.pad(src.reshape(_NW, _EPT), ((0, 0), (0, _EPTP - _EPT)),
                   constant_values=0).reshape(_NW, _NGRP, _NBUF, _KA)
    dst3 = jnp.pad(dst.reshape(_NW, _EPT), ((0, 0), (0, _EPTP - _EPT)),
                   constant_values=_N).reshape(_NW, _NGRP, _NBUF, _KA)

    c0, c1 = _deg_kernel(dst)

    a1 = _mm_first(x, W1, c0, c1)
    q0, q1 = _agg_kernel(a1, src3, dst3)
    a2 = _mm_mid(q0, q1, a1, c0, c1, b1.reshape(1, _H), W2)
    q0, q1 = _agg_kernel(a2, src3, dst3)
    a3 = _mm_mid(q0, q1, a2, c0, c1, b2.reshape(1, _H), W3)
    q0, q1 = _agg_kernel(a3, src3, dst3)

    cpadT = (jnp.zeros((_H, 128), jnp.float32)
             .at[:, :_NC].set(centroids[:, 0, :].T)
             .at[:, 64:64 + _NC].set(centroids[:, 1, :].T))
    max_ac = running_mean + jnp.clip(jnp.maximum(std_scale, 0.0), 0.0, 5.0) * jnp.sqrt(running_var)
    mx_row = jnp.full((1, _H), max_ac, jnp.float32)
    it_row = jnp.full((1, _H), 1.0 / ac_temp, jnp.float32)
    batch3 = batch.reshape(_NBLK, 1, _CH)

    o = _pool_head(q0, q1, a3, c0, c1, b3.reshape(1, _H), batch3, cpadT,
                   mx_row, it_row)
    return o[:, :_NC], o[:, _NC]
